# baseline (speedup 1.0000x reference)
import jax
import jax.numpy as jnp
from jax.experimental import pallas as pl
from jax.experimental.pallas import tpu as pltpu


def _gcn_fused_kernel(adj_ref, x_ref, w0_ref, b0_ref, w1_ref, b1_ref,
                      g_ref, beta_ref, out_ref, mask_ref):
    a = adj_ref[0]
    x = x_ref[0]
    l, d = x.shape
    cs = jnp.sum(a, axis=0, keepdims=True)
    ab = a.astype(jnp.bfloat16)

    xa = jnp.concatenate(
        [x.astype(jnp.bfloat16), jnp.ones((l, d), jnp.bfloat16)], axis=1)
    hf = jnp.dot(ab, xa, preferred_element_type=jnp.float32)
    rs = hf[:, d:d + 1]
    denom = rs + 1.0
    h = hf[:, :d] + x
    h = jnp.dot(h, w0_ref[...], preferred_element_type=jnp.float32) + 2.0 * b0_ref[...]
    h = jax.nn.relu(h / denom)

    h2 = jnp.dot(ab, h.astype(jnp.bfloat16),
                 preferred_element_type=jnp.float32) + h
    h2 = jnp.dot(h2, w1_ref[...], preferred_element_type=jnp.float32) + 2.0 * b1_ref[...]
    h2 = jax.nn.relu(h2 / denom)

    mu = jnp.mean(h2, axis=-1, keepdims=True)
    var = jnp.mean((h2 - mu) * (h2 - mu), axis=-1, keepdims=True)
    y = (h2 - mu) * jax.lax.rsqrt(var + 1e-5) * g_ref[...] + beta_ref[...]

    out_ref[0] = y
    mask_ref[0] = (rs + jnp.transpose(cs)) == 0.0


def kernel(adj, input_emb, seq_lens, W0, b0, W1, b1, ln_gamma, ln_beta):
    B, L, _ = adj.shape
    D = W0.shape[0]
    x0 = input_emb.reshape(B, L, D)
    b0r = b0.reshape(1, D)
    b1r = b1.reshape(1, D)
    gr = ln_gamma.reshape(1, D)
    br = ln_beta.reshape(1, D)

    out, masks = pl.pallas_call(
        _gcn_fused_kernel,
        grid=(B,),
        in_specs=[
            pl.BlockSpec((1, L, L), lambda b: (b, 0, 0)),
            pl.BlockSpec((1, L, D), lambda b: (b, 0, 0)),
            pl.BlockSpec((D, D), lambda b: (0, 0)),
            pl.BlockSpec((1, D), lambda b: (0, 0)),
            pl.BlockSpec((D, D), lambda b: (0, 0)),
            pl.BlockSpec((1, D), lambda b: (0, 0)),
            pl.BlockSpec((1, D), lambda b: (0, 0)),
            pl.BlockSpec((1, D), lambda b: (0, 0)),
        ],
        out_specs=[
            pl.BlockSpec((1, L, D), lambda b: (b, 0, 0)),
            pl.BlockSpec((1, L, 1), lambda b: (b, 0, 0)),
        ],
        out_shape=[
            jax.ShapeDtypeStruct((B, L, D), jnp.float32),
            jax.ShapeDtypeStruct((B, L, 1), jnp.bool_),
        ],
        compiler_params=pltpu.CompilerParams(
            dimension_semantics=("parallel",)),
    )(adj, x0, W0, b0r, W1, b1r, gr, br)

    return (out, masks)


# speedup vs baseline: 1.0786x; 1.0786x over previous
import jax
import jax.numpy as jnp
from jax.experimental import pallas as pl
from jax.experimental.pallas import tpu as pltpu


def _gcn_fused_kernel(adj_ref, x_ref, w0_ref, b0_ref, w1_ref, b1_ref,
                      g_ref, beta_ref, out_ref, mask_ref):
    a = adj_ref[0]
    x = x_ref[0]
    l, d = x.shape
    cs = jnp.sum(a, axis=0, keepdims=True)
    ab = a.astype(jnp.bfloat16)

    xa = jnp.concatenate(
        [x.astype(jnp.bfloat16), jnp.ones((l, d), jnp.bfloat16)], axis=1)
    hf = jnp.dot(ab, xa, preferred_element_type=jnp.float32)
    rs = hf[:, d:d + 1]
    denom = rs + 1.0
    h = hf[:, :d] + x
    h = jnp.dot(h, w0_ref[...], preferred_element_type=jnp.float32) + 2.0 * b0_ref[...]
    h = jax.nn.relu(h / denom)

    h2 = jnp.dot(ab, h.astype(jnp.bfloat16),
                 preferred_element_type=jnp.float32) + h
    h2 = jnp.dot(h2, w1_ref[...], preferred_element_type=jnp.float32) + 2.0 * b1_ref[...]
    h2 = jax.nn.relu(h2 / denom)

    mu = jnp.mean(h2, axis=-1, keepdims=True)
    var = jnp.mean((h2 - mu) * (h2 - mu), axis=-1, keepdims=True)
    y = (h2 - mu) * jax.lax.rsqrt(var + 1e-5) * g_ref[...] + beta_ref[...]

    out_ref[0] = y
    mask_ref[0] = (jnp.transpose(rs) + cs) == 0.0


def kernel(adj, input_emb, seq_lens, W0, b0, W1, b1, ln_gamma, ln_beta):
    B, L, _ = adj.shape
    D = W0.shape[0]
    x0 = input_emb.reshape(B, L, D)
    b0r = b0.reshape(1, D)
    b1r = b1.reshape(1, D)
    gr = ln_gamma.reshape(1, D)
    br = ln_beta.reshape(1, D)

    out, mask_row = pl.pallas_call(
        _gcn_fused_kernel,
        grid=(B,),
        in_specs=[
            pl.BlockSpec((1, L, L), lambda b: (b, 0, 0)),
            pl.BlockSpec((1, L, D), lambda b: (b, 0, 0)),
            pl.BlockSpec((D, D), lambda b: (0, 0)),
            pl.BlockSpec((1, D), lambda b: (0, 0)),
            pl.BlockSpec((D, D), lambda b: (0, 0)),
            pl.BlockSpec((1, D), lambda b: (0, 0)),
            pl.BlockSpec((1, D), lambda b: (0, 0)),
            pl.BlockSpec((1, D), lambda b: (0, 0)),
        ],
        out_specs=[
            pl.BlockSpec((1, L, D), lambda b: (b, 0, 0)),
            pl.BlockSpec((1, 1, L), lambda b: (b, 0, 0)),
        ],
        out_shape=[
            jax.ShapeDtypeStruct((B, L, D), jnp.float32),
            jax.ShapeDtypeStruct((B, 1, L), jnp.bool_),
        ],
        compiler_params=pltpu.CompilerParams(
            dimension_semantics=("parallel",)),
    )(adj, x0, W0, b0r, W1, b1r, gr, br)

    return (out, mask_row.reshape(B, L, 1))


# speedup vs baseline: 1.0794x; 1.0008x over previous
import jax
import jax.numpy as jnp
from jax.experimental import pallas as pl
from jax.experimental.pallas import tpu as pltpu


def _gcn_fused_kernel(adj_ref, x_ref, w0_ref, b0_ref, w1_ref, b1_ref,
                      g_ref, beta_ref, out_ref, mask_ref):
    a = adj_ref[0]
    x = x_ref[0]
    l, d = x.shape
    cs = jnp.sum(a, axis=0, keepdims=True)
    ab = a.astype(jnp.bfloat16)

    xa = jnp.concatenate(
        [x.astype(jnp.bfloat16), jnp.ones((l, d), jnp.bfloat16)], axis=1)
    hf = jnp.dot(ab, xa, preferred_element_type=jnp.float32)
    rs = hf[:, d:d + 1]
    denom = rs + 1.0
    h = hf[:, :d] + x
    h = jnp.dot(h, w0_ref[...], preferred_element_type=jnp.float32) + 2.0 * b0_ref[...]
    h = jax.nn.relu(h / denom)

    h2 = jnp.dot(ab, h.astype(jnp.bfloat16),
                 preferred_element_type=jnp.float32) + h
    h2 = jnp.dot(h2, w1_ref[...], preferred_element_type=jnp.float32) + 2.0 * b1_ref[...]
    h2 = jax.nn.relu(h2 / denom)

    mu = jnp.mean(h2, axis=-1, keepdims=True)
    var = jnp.mean((h2 - mu) * (h2 - mu), axis=-1, keepdims=True)
    y = (h2 - mu) * jax.lax.rsqrt(var + 1e-5) * g_ref[...] + beta_ref[...]

    out_ref[0] = y
    mask_ref[0] = (jnp.transpose(rs) + cs) == 0.0


def kernel(adj, input_emb, seq_lens, W0, b0, W1, b1, ln_gamma, ln_beta):
    B, L, _ = adj.shape
    D = W0.shape[0]
    x0 = input_emb.reshape(B, L, D)
    b0r = b0.reshape(1, D)
    b1r = b1.reshape(1, D)
    gr = ln_gamma.reshape(1, D)
    br = ln_beta.reshape(1, D)

    out, mask_row = pl.pallas_call(
        _gcn_fused_kernel,
        grid=(B,),
        in_specs=[
            pl.BlockSpec((1, L, L), lambda b: (b, 0, 0)),
            pl.BlockSpec((1, L, D), lambda b: (b, 0, 0)),
            pl.BlockSpec((D, D), lambda b: (0, 0)),
            pl.BlockSpec((1, D), lambda b: (0, 0)),
            pl.BlockSpec((D, D), lambda b: (0, 0)),
            pl.BlockSpec((1, D), lambda b: (0, 0)),
            pl.BlockSpec((1, D), lambda b: (0, 0)),
            pl.BlockSpec((1, D), lambda b: (0, 0)),
        ],
        out_specs=[
            pl.BlockSpec((1, L, D), lambda b: (b, 0, 0)),
            pl.BlockSpec((1, 1, L), lambda b: (b, 0, 0)),
        ],
        out_shape=[
            jax.ShapeDtypeStruct((B, L, D), jnp.float32),
            jax.ShapeDtypeStruct((B, 1, L), jnp.bool_),
        ],
        compiler_params=pltpu.CompilerParams(
            dimension_semantics=("arbitrary",)),
    )(adj, x0, W0, b0r, W1, b1r, gr, br)

    return (out, mask_row.reshape(B, L, 1))
